# Initial kernel scaffold; baseline (speedup 1.0000x reference)
import jax
import jax.numpy as jnp
from jax.experimental import pallas as pl


def kernel(feats, feat_depths, novel_views):
    raise NotImplementedError("write your pallas kernel here")


# speedup vs baseline: 2.2412x; 2.2412x over previous
import functools

import jax
import jax.numpy as jnp
import numpy as np
from jax import lax
from jax.experimental import pallas as pl
from jax.experimental.pallas import tpu as pltpu
from jax.experimental.pallas import tpu_sc as plsc

_FEAT_DIM = 256
_F_CAM = 0.02
_DIOPTERS = np.array([0.0, 1.0, 2.0, 3.0], dtype=np.float32)
_VIEW_POS = np.array(
    [[x, y] for y in (-0.5, 0.0, 0.5) for x in (-0.5, 0.0, 0.5)],
    dtype=np.float32,
)

_H = 256
_W = 256
_PAD = 32
_HP = _H + 2 * _PAD
_WP = _W + 2 * _PAD
_NV = 9
_NN = 2
_C = 32
_NL = 4
_PLANE = _H * _W
_DPLANE = _HP * _WP
_SENT = _PLANE
_LANES = 16
_STRIP_ROWS = 32
_STRIP = _STRIP_ROWS * _W
_NSTRIP = _H // _STRIP_ROWS
_VPR = _W // _LANES


def _sc_body(feats_hbm, depth_hbm, offd_hbm, offo_hbm, out_hbm, idx_hbm,
             planeA, idxB, outC, offd_v, offo_v):
    c = lax.axis_index("c")
    s = lax.axis_index("s")
    iota = lax.iota(jnp.int32, _LANES)

    @pl.when(s < _NV)
    def _phase1():
        p = c * _NV + s
        pltpu.sync_copy(depth_hbm.at[s], planeA.at[pl.ds(0, _DPLANE)])
        pltpu.sync_copy(offd_hbm.at[p], offd_v)
        pltpu.sync_copy(offo_hbm.at[p], offo_v)
        offd = [offd_v[l] for l in range(_NL)]
        offo = [offo_v[l] for l in range(_NL)]

        def strip_body(r, carry):
            def row_body(rr, carry):
                y = r * _STRIP_ROWS + rr

                def vec_body(j, carry):
                    x0 = j * _LANES
                    base_d = (y + _PAD) * _WP + (x0 + _PAD) + iota
                    base_o = y * _W + x0 + iota
                    idx = jnp.full((_LANES,), _SENT, dtype=jnp.int32)
                    for l in range(_NL):
                        dl = plsc.load_gather(planeA, [base_d - offd[l]])
                        idx = jnp.where(dl == np.float32(l),
                                        base_o - offo[l], idx)
                    idxB[pl.ds(rr * _W + x0, _LANES)] = idx
                    return carry

                return lax.fori_loop(0, _VPR, vec_body, carry)

            lax.fori_loop(0, _STRIP_ROWS, row_body, 0)
            pltpu.sync_copy(idxB, idx_hbm.at[p, pl.ds(r * _STRIP, _STRIP)])
            return carry

        lax.fori_loop(0, _NSTRIP, strip_body, 0)

    plsc.subcore_barrier()

    planeA[pl.ds(_PLANE, _LANES)] = jnp.zeros((_LANES,), jnp.float32)
    for k in range(_NV * _C // _LANES):
        t = k * _LANES + s
        v = t // _C
        ch = t % _C
        p = c * _NV + v
        frow = v * _C + ch
        orow = p * _C + ch
        pltpu.sync_copy(feats_hbm.at[frow], planeA.at[pl.ds(0, _PLANE)])

        def strip2(r, carry):
            pltpu.sync_copy(idx_hbm.at[p, pl.ds(r * _STRIP, _STRIP)], idxB)

            def vec2(j, carry):
                o = j * _LANES
                iv = idxB[pl.ds(o, _LANES)]
                outC[pl.ds(o, _LANES)] = plsc.load_gather(planeA, [iv])
                return carry

            lax.fori_loop(0, _STRIP // _LANES, vec2, carry)
            pltpu.sync_copy(outC, out_hbm.at[orow, pl.ds(r * _STRIP, _STRIP)])
            return carry

        lax.fori_loop(0, _NSTRIP, strip2, 0)


_sc_call = pl.kernel(
    _sc_body,
    out_type=[
        jax.ShapeDtypeStruct((_NN * _NV * _C, _PLANE), jnp.float32),
        jax.ShapeDtypeStruct((_NN * _NV, _PLANE), jnp.int32),
    ],
    mesh=plsc.VectorSubcoreMesh(core_axis_name="c", subcore_axis_name="s"),
    compiler_params=pltpu.CompilerParams(needs_layout_passes=False),
    scratch_types=[
        pltpu.VMEM((_DPLANE,), jnp.float32),
        pltpu.VMEM((_STRIP,), jnp.int32),
        pltpu.VMEM((_STRIP,), jnp.float32),
        pltpu.VMEM((_NL, _LANES), jnp.int32),
        pltpu.VMEM((_NL, _LANES), jnp.int32),
    ],
)


def kernel(feats, feat_depths, novel_views):
    nn = novel_views.shape[0]
    nv, C, H, W = feats.shape

    nvw = novel_views[:, None, None, :]
    vp = jnp.asarray(_VIEW_POS)[None, :, None, :]
    dio = jnp.asarray(_DIOPTERS)[None, None, :, None]
    d = jnp.round((vp - nvw) * dio * np.float32(_F_CAM * _FEAT_DIM))
    d = d.astype(jnp.int32)
    dx = d[..., 0]
    dy = d[..., 1]
    offd = (dy * _WP + dx).reshape(nn * nv, _NL)
    offo = (dy * _W + dx).reshape(nn * nv, _NL)
    offd16 = jnp.broadcast_to(offd[:, :, None], (nn * nv, _NL, _LANES))
    offo16 = jnp.broadcast_to(offo[:, :, None], (nn * nv, _NL, _LANES))
    offd16 = jnp.asarray(offd16, dtype=jnp.int32)
    offo16 = jnp.asarray(offo16, dtype=jnp.int32)

    depth = feat_depths[:, 0]
    depth_pad = jnp.pad(depth, ((0, 0), (_PAD, _PAD), (_PAD, _PAD)),
                        constant_values=_NL)
    depth_pad = depth_pad.astype(jnp.float32).reshape(nv, _DPLANE)

    feats_flat = feats.reshape(nv * C, H * W)

    out, _ = _sc_call(feats_flat, depth_pad, offd16, offo16)
    return out.reshape(nn, nv, C, H, W)


# speedup vs baseline: 3.4865x; 1.5556x over previous
import functools

import jax
import jax.numpy as jnp
import numpy as np
from jax import lax
from jax.experimental import pallas as pl
from jax.experimental.pallas import tpu as pltpu
from jax.experimental.pallas import tpu_sc as plsc

_FEAT_DIM = 256
_F_CAM = 0.02
_DIOPTERS = np.array([0.0, 1.0, 2.0, 3.0], dtype=np.float32)
_VIEW_POS = np.array(
    [[x, y] for y in (-0.5, 0.0, 0.5) for x in (-0.5, 0.0, 0.5)],
    dtype=np.float32,
)

_H = 256
_W = 256
_PAD = 32
_HP = _H + 2 * _PAD
_WP = _W + 2 * _PAD
_NV = 9
_NN = 2
_C = 32
_NL = 4
_PLANE = _H * _W
_DPLANE = _HP * _WP
_SENT = _PLANE
_LANES = 16
_STRIP_ROWS = 32
_STRIP = _STRIP_ROWS * _W
_NSTRIP = _H // _STRIP_ROWS
_VPR = _W // _LANES


def _sc_body(feats_hbm, depth_hbm, offd_hbm, offo_hbm, out_hbm, idx_hbm,
             planeA, idxB, outC, offd_v, offo_v):
    c = lax.axis_index("c")
    s = lax.axis_index("s")
    iota = lax.iota(jnp.int32, _LANES)

    @pl.when(s < _NV)
    def _phase1():
        p = c * _NV + s
        pltpu.sync_copy(depth_hbm.at[s], planeA.at[pl.ds(0, _DPLANE)])
        pltpu.sync_copy(offd_hbm.at[p], offd_v)
        pltpu.sync_copy(offo_hbm.at[p], offo_v)
        offd = [offd_v[l] for l in range(_NL)]
        offo = [offo_v[l] for l in range(_NL)]

        def strip_body(r, carry):
            y0 = r * _STRIP_ROWS

            @plsc.parallel_loop(0, _STRIP_ROWS * _VPR, unroll=4)
            def _p1_vec(j):
                rr = j // _VPR
                x0 = (j % _VPR) * _LANES
                y = y0 + rr
                base_d = (y + _PAD) * _WP + (x0 + _PAD) + iota
                base_o = y * _W + x0 + iota
                idx = jnp.full((_LANES,), _SENT, dtype=jnp.int32)
                for l in range(_NL):
                    dl = plsc.load_gather(planeA, [base_d - offd[l]])
                    idx = jnp.where(dl == np.float32(l),
                                    base_o - offo[l], idx)
                idxB[pl.ds(rr * _W + x0, _LANES)] = idx
            pltpu.sync_copy(idxB, idx_hbm.at[p, pl.ds(r * _STRIP, _STRIP)])
            return carry

        lax.fori_loop(0, _NSTRIP, strip_body, 0)

    plsc.subcore_barrier()

    planeA[pl.ds(_PLANE, _LANES)] = jnp.zeros((_LANES,), jnp.float32)
    for k in range(_NV * _C // _LANES):
        t = k * _LANES + s
        v = t // _C
        ch = t % _C
        p = c * _NV + v
        frow = v * _C + ch
        orow = p * _C + ch
        pltpu.sync_copy(feats_hbm.at[frow], planeA.at[pl.ds(0, _PLANE)])

        def strip2(r, carry):
            pltpu.sync_copy(idx_hbm.at[p, pl.ds(r * _STRIP, _STRIP)], idxB)

            @plsc.parallel_loop(0, _STRIP // _LANES, unroll=8)
            def _vec2(j):
                o = j * _LANES
                iv = idxB[pl.ds(o, _LANES)]
                outC[pl.ds(o, _LANES)] = plsc.load_gather(planeA, [iv])
            pltpu.sync_copy(outC, out_hbm.at[orow, pl.ds(r * _STRIP, _STRIP)])
            return carry

        lax.fori_loop(0, _NSTRIP, strip2, 0)


_sc_call = pl.kernel(
    _sc_body,
    out_type=[
        jax.ShapeDtypeStruct((_NN * _NV * _C, _PLANE), jnp.float32),
        jax.ShapeDtypeStruct((_NN * _NV, _PLANE), jnp.int32),
    ],
    mesh=plsc.VectorSubcoreMesh(core_axis_name="c", subcore_axis_name="s"),
    compiler_params=pltpu.CompilerParams(needs_layout_passes=False),
    scratch_types=[
        pltpu.VMEM((_DPLANE,), jnp.float32),
        pltpu.VMEM((_STRIP,), jnp.int32),
        pltpu.VMEM((_STRIP,), jnp.float32),
        pltpu.VMEM((_NL, _LANES), jnp.int32),
        pltpu.VMEM((_NL, _LANES), jnp.int32),
    ],
)


def kernel(feats, feat_depths, novel_views):
    nn = novel_views.shape[0]
    nv, C, H, W = feats.shape

    nvw = novel_views[:, None, None, :]
    vp = jnp.asarray(_VIEW_POS)[None, :, None, :]
    dio = jnp.asarray(_DIOPTERS)[None, None, :, None]
    d = jnp.round((vp - nvw) * dio * np.float32(_F_CAM * _FEAT_DIM))
    d = d.astype(jnp.int32)
    dx = d[..., 0]
    dy = d[..., 1]
    offd = (dy * _WP + dx).reshape(nn * nv, _NL)
    offo = (dy * _W + dx).reshape(nn * nv, _NL)
    offd16 = jnp.broadcast_to(offd[:, :, None], (nn * nv, _NL, _LANES))
    offo16 = jnp.broadcast_to(offo[:, :, None], (nn * nv, _NL, _LANES))
    offd16 = jnp.asarray(offd16, dtype=jnp.int32)
    offo16 = jnp.asarray(offo16, dtype=jnp.int32)

    depth = feat_depths[:, 0]
    depth_pad = jnp.pad(depth, ((0, 0), (_PAD, _PAD), (_PAD, _PAD)),
                        constant_values=_NL)
    depth_pad = depth_pad.astype(jnp.float32).reshape(nv, _DPLANE)

    feats_flat = feats.reshape(nv * C, H * W)

    out, _ = _sc_call(feats_flat, depth_pad, offd16, offo16)
    return out.reshape(nn, nv, C, H, W)


# speedup vs baseline: 4.6828x; 1.3432x over previous
import functools

import jax
import jax.numpy as jnp
import numpy as np
from jax import lax
from jax.experimental import pallas as pl
from jax.experimental.pallas import tpu as pltpu
from jax.experimental.pallas import tpu_sc as plsc

_FEAT_DIM = 256
_F_CAM = 0.02
_DIOPTERS = np.array([0.0, 1.0, 2.0, 3.0], dtype=np.float32)
_VIEW_POS = np.array(
    [[x, y] for y in (-0.5, 0.0, 0.5) for x in (-0.5, 0.0, 0.5)],
    dtype=np.float32,
)

_H = 256
_W = 256
_PAD = 32
_HP = _H + 2 * _PAD
_WP = _W + 2 * _PAD
_NV = 9
_NN = 2
_C = 32
_NL = 4
_PLANE = _H * _W
_DPLANE = _HP * _WP
_SENT = _PLANE
_LANES = 16
_STRIP_ROWS = 32
_STRIP = _STRIP_ROWS * _W
_NSTRIP = _H // _STRIP_ROWS
_VPR = _W // _LANES
_HALO = (_STRIP_ROWS + 2 * _PAD) * _WP


def _sc_body(feats_hbm, depth_hbm, offd_hbm, offo_hbm, out_hbm, idx_hbm,
             planeA, haloD, idxB, outC, offd_v, offo_v,
             isem0, isem1, osem0, osem1):
    c = lax.axis_index("c")
    s = lax.axis_index("s")
    iota = lax.iota(jnp.int32, _LANES)
    isems = (isem0, isem1)
    osems = (osem0, osem1)

    @pl.when(s < _NV)
    def _phase1():
        p = c * _NV + s
        pltpu.sync_copy(offd_hbm.at[p], offd_v)
        pltpu.sync_copy(offo_hbm.at[p], offo_v)
        offd = [offd_v[l] for l in range(_NL)]
        offo = [offo_v[l] for l in range(_NL)]

        def strip_body(r, carry):
            pltpu.sync_copy(
                depth_hbm.at[s, pl.ds(r * _STRIP_ROWS * _WP, _HALO)], haloD)
            y0 = r * _STRIP_ROWS

            @plsc.parallel_loop(0, _STRIP_ROWS * _VPR, unroll=4)
            def _p1_vec(j):
                rr = j // _VPR
                x0 = (j % _VPR) * _LANES
                base_d = (rr + _PAD) * _WP + (x0 + _PAD) + iota
                base_o = (y0 + rr) * _W + x0 + iota
                idx = jnp.full((_LANES,), _SENT, dtype=jnp.int32)
                for l in range(_NL):
                    dl = plsc.load_gather(haloD, [base_d - offd[l]])
                    idx = jnp.where(dl == np.float32(l),
                                    base_o - offo[l], idx)
                idxB[pl.ds(rr * _W + x0, _LANES)] = idx

            pltpu.sync_copy(idxB.at[pl.ds(0, _STRIP)],
                            idx_hbm.at[p, pl.ds(r * _STRIP, _STRIP)])
            return carry

        lax.fori_loop(0, _NSTRIP, strip_body, 0)

    plsc.subcore_barrier()

    planeA[pl.ds(_PLANE, _LANES)] = jnp.zeros((_LANES,), jnp.float32)

    def _idx_cp(p, r):
        b = r % 2
        return pltpu.make_async_copy(
            idx_hbm.at[p, pl.ds(r * _STRIP, _STRIP)],
            idxB.at[pl.ds(b * _STRIP, _STRIP)], isems[b])

    def _out_cp(orow, r):
        b = r % 2
        return pltpu.make_async_copy(
            outC.at[pl.ds(b * _STRIP, _STRIP)],
            out_hbm.at[orow, pl.ds(r * _STRIP, _STRIP)], osems[b])

    def run_task(k, first):
        t = k * _LANES + s
        v = t // _C
        ch = t % _C
        p = c * _NV + v
        frow = v * _C + ch
        orow = p * _C + ch
        _idx_cp(p, 0).start()
        pltpu.sync_copy(feats_hbm.at[frow], planeA.at[pl.ds(0, _PLANE)])
        for r in range(_NSTRIP):
            b = r % 2
            if r + 1 < _NSTRIP:
                _idx_cp(p, r + 1).start()
            _idx_cp(p, r).wait()
            if not (first and r < 2):
                _out_cp(orow, r).wait()

            @plsc.parallel_loop(0, _STRIP // _LANES, unroll=16)
            def _vec2(j):
                o = b * _STRIP + j * _LANES
                iv = idxB[pl.ds(o, _LANES)]
                outC[pl.ds(o, _LANES)] = plsc.load_gather(planeA, [iv])

            _out_cp(orow, r).start()

    run_task(0, True)

    def task_body(k, carry):
        run_task(k, False)
        return carry

    lax.fori_loop(1, _NV * _C // _LANES, task_body, 0)

    for r in (_NSTRIP - 2, _NSTRIP - 1):
        _out_cp(0, r).wait()


_sc_call = pl.kernel(
    _sc_body,
    out_type=[
        jax.ShapeDtypeStruct((_NN * _NV * _C, _PLANE), jnp.float32),
        jax.ShapeDtypeStruct((_NN * _NV, _PLANE), jnp.int32),
    ],
    mesh=plsc.VectorSubcoreMesh(core_axis_name="c", subcore_axis_name="s"),
    compiler_params=pltpu.CompilerParams(needs_layout_passes=False),
    scratch_types=[
        pltpu.VMEM((_PLANE + _LANES,), jnp.float32),
        pltpu.VMEM((_HALO,), jnp.float32),
        pltpu.VMEM((2 * _STRIP,), jnp.int32),
        pltpu.VMEM((2 * _STRIP,), jnp.float32),
        pltpu.VMEM((_NL, _LANES), jnp.int32),
        pltpu.VMEM((_NL, _LANES), jnp.int32),
        pltpu.SemaphoreType.DMA,
        pltpu.SemaphoreType.DMA,
        pltpu.SemaphoreType.DMA,
        pltpu.SemaphoreType.DMA,
    ],
)


def kernel(feats, feat_depths, novel_views):
    nn = novel_views.shape[0]
    nv, C, H, W = feats.shape

    nvw = novel_views[:, None, None, :]
    vp = jnp.asarray(_VIEW_POS)[None, :, None, :]
    dio = jnp.asarray(_DIOPTERS)[None, None, :, None]
    d = jnp.round((vp - nvw) * dio * np.float32(_F_CAM * _FEAT_DIM))
    d = d.astype(jnp.int32)
    dx = d[..., 0]
    dy = d[..., 1]
    offd = (dy * _WP + dx).reshape(nn * nv, _NL)
    offo = (dy * _W + dx).reshape(nn * nv, _NL)
    offd16 = jnp.broadcast_to(offd[:, :, None], (nn * nv, _NL, _LANES))
    offo16 = jnp.broadcast_to(offo[:, :, None], (nn * nv, _NL, _LANES))
    offd16 = jnp.asarray(offd16, dtype=jnp.int32)
    offo16 = jnp.asarray(offo16, dtype=jnp.int32)

    depth = feat_depths[:, 0]
    depth_pad = jnp.pad(depth, ((0, 0), (_PAD, _PAD), (_PAD, _PAD)),
                        constant_values=_NL)
    depth_pad = depth_pad.astype(jnp.float32).reshape(nv, _DPLANE)

    feats_flat = feats.reshape(nv * C, H * W)

    out, _ = _sc_call(feats_flat, depth_pad, offd16, offo16)
    return out.reshape(nn, nv, C, H, W)


# speedup vs baseline: 4.9999x; 1.0677x over previous
import functools

import jax
import jax.numpy as jnp
import numpy as np
from jax import lax
from jax.experimental import pallas as pl
from jax.experimental.pallas import tpu as pltpu
from jax.experimental.pallas import tpu_sc as plsc

_FEAT_DIM = 256
_F_CAM = 0.02
_DIOPTERS = np.array([0.0, 1.0, 2.0, 3.0], dtype=np.float32)
_VIEW_POS = np.array(
    [[x, y] for y in (-0.5, 0.0, 0.5) for x in (-0.5, 0.0, 0.5)],
    dtype=np.float32,
)

_H = 256
_W = 256
_PAD = 32
_HP = _H + 2 * _PAD
_WP = _W + 2 * _PAD
_NV = 9
_NN = 2
_C = 32
_NL = 4
_PLANE = _H * _W
_DPLANE = _HP * _WP
_LANES = 16
_STRIP_ROWS = 32
_STRIP = _STRIP_ROWS * _W
_NSTRIP = _H // _STRIP_ROWS
_VPR = _W // _LANES
_HALO = (_STRIP_ROWS + 2 * _PAD) * _WP
_NTASK = _NV * _C // _LANES
_NG = _NTASK * _NSTRIP
_ARENA = 4 * _STRIP
_ZSLOT = _ARENA
_P1T = _NV * _NSTRIP


def _sc_body(feats_hbm, depth_hbm, offd_hbm, offo_hbm, out_hbm, idx_hbm,
             arena, haloD, idxB, outC, offd_v, offo_v,
             psem0, psem1, psem2, psem3, isem0, isem1, osem0, osem1):
    c = lax.axis_index("c")
    s = lax.axis_index("s")
    iota = lax.iota(jnp.int32, _LANES)
    psems = (psem0, psem1, psem2, psem3)
    isems = (isem0, isem1)
    osems = (osem0, osem1)

    def p1_task(m, carry):
        T = m * _LANES + s

        @pl.when(T < _P1T)
        def _():
            pv = T // _NSTRIP
            r = T % _NSTRIP
            p = c * _NV + pv
            pltpu.sync_copy(offd_hbm.at[p], offd_v)
            pltpu.sync_copy(offo_hbm.at[p], offo_v)
            pltpu.sync_copy(
                depth_hbm.at[pv, pl.ds(r * _STRIP_ROWS * _WP, _HALO)], haloD)
            offd = [offd_v[l] for l in range(_NL)]
            offo = [offo_v[l] for l in range(_NL)]
            y0 = r * _STRIP_ROWS

            @plsc.parallel_loop(0, _STRIP_ROWS * _VPR, unroll=4)
            def _p1_vec(j):
                rr = j // _VPR
                x0 = (j % _VPR) * _LANES
                base_d = (rr + _PAD) * _WP + (x0 + _PAD) + iota
                base_o = (y0 + rr) * _W + x0 + iota
                idx = jnp.full((_LANES,), -1, dtype=jnp.int32)
                for l in range(_NL):
                    dl = plsc.load_gather(haloD, [base_d - offd[l]])
                    idx = jnp.where(dl == np.float32(l),
                                    base_o - offo[l], idx)
                idxB[pl.ds(rr * _W + x0, _LANES)] = idx

            pltpu.sync_copy(idxB.at[pl.ds(0, _STRIP)],
                            idx_hbm.at[p, pl.ds(r * _STRIP, _STRIP)])

        return carry

    lax.fori_loop(0, (_P1T + _LANES - 1) // _LANES, p1_task, 0)

    plsc.subcore_barrier()

    arena[pl.ds(_ZSLOT, _LANES)] = jnp.zeros((_LANES,), jnp.float32)

    def _rows(g):
        k = g // _NSTRIP
        t = k * _LANES + s
        v = t // _C
        ch = t % _C
        return v * _C + ch, (c * _NV + v) * _C + ch, c * _NV + v

    def _pl_cp(g, slot):
        frow, _, _ = _rows(g)
        r = g % _NSTRIP
        return pltpu.make_async_copy(
            feats_hbm.at[frow, pl.ds(r * _STRIP, _STRIP)],
            arena.at[pl.ds(slot * _STRIP, _STRIP)], psems[slot])

    def _idx_cp(g, b):
        _, _, prow = _rows(g)
        r = g % _NSTRIP
        return pltpu.make_async_copy(
            idx_hbm.at[prow, pl.ds(r * _STRIP, _STRIP)],
            idxB.at[pl.ds(b * _STRIP, _STRIP)], isems[b])

    def _out_cp(g, b):
        _, orow, _ = _rows(g)
        r = g % _NSTRIP
        return pltpu.make_async_copy(
            outC.at[pl.ds(b * _STRIP, _STRIP)],
            out_hbm.at[orow, pl.ds(r * _STRIP, _STRIP)], osems[b])

    def _gather(b):

        @plsc.parallel_loop(0, _STRIP // _LANES, unroll=16)
        def _vec(j):
            o = j * _LANES
            iv = idxB[pl.ds(b * _STRIP + o, _LANES)]
            loc = jnp.where(iv < 0, _ZSLOT,
                            jnp.bitwise_and(iv, _ARENA - 1))
            outC[pl.ds(b * _STRIP + o, _LANES)] = \
                plsc.load_gather(arena, [loc])

    _pl_cp(0, 0).start()
    _pl_cp(1, 1).start()
    _idx_cp(0, 0).start()
    _idx_cp(1, 1).start()
    _pl_cp(0, 0).wait()

    for g in range(4):
        _idx_cp(g, g % 2).wait()
        _pl_cp(g + 1, (g + 1) % 4).wait()
        _pl_cp(g + 2, (g + 2) % 4).start()
        if g >= 2:
            _out_cp(g, g % 2).wait()
        _gather(g % 2)
        _idx_cp(g + 2, g % 2).start()
        _out_cp(g, g % 2).start()

    def quad(q, carry):
        g0 = q * 4
        for j in range(4):
            g = g0 + j
            _idx_cp(g, j % 2).wait()
            _pl_cp(g + 1, (j + 1) % 4).wait()
            _pl_cp(g + 2, (j + 2) % 4).start()
            _out_cp(g, j % 2).wait()
            _gather(j % 2)
            _idx_cp(g + 2, j % 2).start()
            _out_cp(g, j % 2).start()
        return carry

    lax.fori_loop(1, _NG // 4 - 1, quad, 0)

    for g in range(_NG - 4, _NG):
        _idx_cp(g, g % 2).wait()
        if g + 1 < _NG:
            _pl_cp(g + 1, (g + 1) % 4).wait()
        if g + 2 < _NG:
            _pl_cp(g + 2, (g + 2) % 4).start()
        _out_cp(g, g % 2).wait()
        _gather(g % 2)
        if g + 2 < _NG:
            _idx_cp(g + 2, g % 2).start()
        _out_cp(g, g % 2).start()

    _out_cp(_NG - 2, 0).wait()
    _out_cp(_NG - 1, 1).wait()


_sc_call = pl.kernel(
    _sc_body,
    out_type=[
        jax.ShapeDtypeStruct((_NN * _NV * _C, _PLANE), jnp.float32),
        jax.ShapeDtypeStruct((_NN * _NV, _PLANE), jnp.int32),
    ],
    mesh=plsc.VectorSubcoreMesh(core_axis_name="c", subcore_axis_name="s"),
    compiler_params=pltpu.CompilerParams(needs_layout_passes=False),
    scratch_types=[
        pltpu.VMEM((_ARENA + _LANES,), jnp.float32),
        pltpu.VMEM((_HALO,), jnp.float32),
        pltpu.VMEM((2 * _STRIP,), jnp.int32),
        pltpu.VMEM((2 * _STRIP,), jnp.float32),
        pltpu.VMEM((_NL, _LANES), jnp.int32),
        pltpu.VMEM((_NL, _LANES), jnp.int32),
        pltpu.SemaphoreType.DMA,
        pltpu.SemaphoreType.DMA,
        pltpu.SemaphoreType.DMA,
        pltpu.SemaphoreType.DMA,
        pltpu.SemaphoreType.DMA,
        pltpu.SemaphoreType.DMA,
        pltpu.SemaphoreType.DMA,
        pltpu.SemaphoreType.DMA,
    ],
)


def kernel(feats, feat_depths, novel_views):
    nn = novel_views.shape[0]
    nv, C, H, W = feats.shape

    nvw = novel_views[:, None, None, :]
    vp = jnp.asarray(_VIEW_POS)[None, :, None, :]
    dio = jnp.asarray(_DIOPTERS)[None, None, :, None]
    d = jnp.round((vp - nvw) * dio * np.float32(_F_CAM * _FEAT_DIM))
    d = d.astype(jnp.int32)
    dx = d[..., 0]
    dy = d[..., 1]
    offd = (dy * _WP + dx).reshape(nn * nv, _NL)
    offo = (dy * _W + dx).reshape(nn * nv, _NL)
    offd16 = jnp.broadcast_to(offd[:, :, None], (nn * nv, _NL, _LANES))
    offo16 = jnp.broadcast_to(offo[:, :, None], (nn * nv, _NL, _LANES))
    offd16 = jnp.asarray(offd16, dtype=jnp.int32)
    offo16 = jnp.asarray(offo16, dtype=jnp.int32)

    depth = feat_depths[:, 0]
    depth_pad = jnp.pad(depth, ((0, 0), (_PAD, _PAD), (_PAD, _PAD)),
                        constant_values=_NL)
    depth_pad = depth_pad.astype(jnp.float32).reshape(nv, _DPLANE)

    feats_flat = feats.reshape(nv * C, H * W)

    out, _ = _sc_call(feats_flat, depth_pad, offd16, offo16)
    return out.reshape(nn, nv, C, H, W)
